# baseline (speedup 1.0000x reference)
import functools

import jax
import jax.numpy as jnp
from jax import lax
from jax.experimental import pallas as pl
from jax.experimental.pallas import tpu as pltpu
from jax.experimental.pallas import tpu_sc as plsc

_N = 10000
_D = 128
_NP = 10240
_BR = 1024
_GRID = _NP // _BR
_EPS = 1e-5

_EC = 128
_CH = 80
_NB = 2
_NI = 2 * _NB
_EPAD = 32 * _CH * _EC
_NROWS = _EPAD // _EC + _NI
_RPS = _NP // 16


def _sc_agg_body(h_hbm, src_hbm, dst_hbm, out_hbm, accum, sidx, didx, rows,
                 sem_g, sem_is, sem_id):
    c = lax.axis_index("c")
    s = lax.axis_index("s")
    wid = s * 2 + c
    base = wid * _CH

    pltpu.sync_copy(h_hbm.at[pl.ds(s * _RPS, _RPS)],
                    accum.at[pl.ds(s * _RPS, _RPS)])

    def _ld_idx(j, sl):
        return (pltpu.make_async_copy(src_hbm.at[base + j], sidx.at[sl],
                                      sem_is.at[sl]),
                pltpu.make_async_copy(dst_hbm.at[base + j], didx.at[sl],
                                      sem_id.at[sl]))

    def _gather(sl, b):
        return pltpu.make_async_copy(h_hbm.at[sidx.at[sl]], rows.at[b],
                                     sem_g.at[b])

    for k in range(_NI):
        scp, dcp = _ld_idx(k, k)
        scp.start()
        dcp.start()
    plsc.subcore_barrier()

    for b in range(_NB):
        scp, _ = _ld_idx(b, b)
        scp.wait()
        _gather(b, b).start()

    def _visit(j, b, last):
        ib = j % _NI
        _gather(ib, b).wait()
        _, dcp = _ld_idx(j, ib)
        dcp.wait()
        pltpu.sync_copy(rows.at[b], accum.at[didx.at[ib]], add=True)
        scp, dcp = _ld_idx(j + _NI, ib)
        scp.start()
        dcp.start()
        if not last:
            jn = j + _NB
            scp2, _ = _ld_idx(jn, jn % _NI)
            scp2.wait()
            _gather(jn % _NI, b).start()

    def outer(grp, carry):
        for b in range(_NB):
            _visit(grp * _NB + b, b, False)
        return carry

    lax.fori_loop(0, _CH // _NB - 1, outer, 0)
    for b in range(_NB):
        _visit((_CH // _NB - 1) * _NB + b, b, True)

    for k in range(_NI):
        jo = _CH + k
        scp, dcp = _ld_idx(jo, jo % _NI)
        scp.wait()
        dcp.wait()

    plsc.subcore_barrier()
    pltpu.sync_copy(accum.at[pl.ds(s * _RPS, _RPS)],
                    out_hbm.at[c, pl.ds(s * _RPS, _RPS)])


def _sc_agg(hp, src2, dst2):
    mesh = plsc.VectorSubcoreMesh(core_axis_name="c", subcore_axis_name="s")
    f = pl.kernel(
        _sc_agg_body, mesh=mesh,
        out_type=jax.ShapeDtypeStruct((2, _NP, _D), jnp.float32),
        scratch_types=[
            pltpu.VMEM_SHARED((_NP, _D), jnp.float32),
            pltpu.VMEM((_NI, _EC), jnp.int32),
            pltpu.VMEM((_NI, _EC), jnp.int32),
            pltpu.VMEM((_NB, _EC, _D), jnp.float32),
            pltpu.SemaphoreType.DMA((_NB,)),
            pltpu.SemaphoreType.DMA((_NI,)),
            pltpu.SemaphoreType.DMA((_NI,)),
        ],
    )
    return f(hp, src2, dst2)


def _mm_stats(X, w_ref, b_ref, y_ref, s_ref, ss_ref):
    Y = lax.dot_general(X, w_ref[...], (((1,), (0,)), ((), ())),
                        precision=lax.Precision.HIGHEST) + b_ref[...]
    y_ref[...] = Y
    i = pl.program_id(0)
    rid = lax.broadcasted_iota(jnp.int32, (_BR, 1), 0) + i * _BR
    Ym = jnp.where(rid < _N, Y, 0.0)

    @pl.when(i == 0)
    def _():
        s_ref[...] = jnp.zeros_like(s_ref)
        ss_ref[...] = jnp.zeros_like(ss_ref)

    s_ref[...] += jnp.sum(Ym, axis=0, keepdims=True)
    ss_ref[...] += jnp.sum(Ym * Ym, axis=0, keepdims=True)


def _k_agg_mm(p_ref, x_ref, w, b, y, s, ss):
    X = p_ref[0] + p_ref[1] - x_ref[...]
    _mm_stats(X, w, b, y, s, ss)


def _k_aff(yin, a, c, w, b, y, s, ss):
    X = jnp.maximum(yin[...] * a[...] + c[...], 0.0)
    _mm_stats(X, w, b, y, s, ss)


def _k_dual(p_ref, x_ref, q_ref, h_ref, wa, wb, b, y_ref, s_ref, ss_ref):
    X1 = p_ref[0] + p_ref[1] - x_ref[...]
    X2 = q_ref[0] + q_ref[1] - h_ref[...]
    Y = (lax.dot_general(X1, wa[...], (((1,), (0,)), ((), ())),
                         precision=lax.Precision.HIGHEST)
         + lax.dot_general(X2, wb[...], (((1,), (0,)), ((), ())),
                           precision=lax.Precision.HIGHEST) + b[...])
    y_ref[...] = Y
    i = pl.program_id(0)
    rid = lax.broadcasted_iota(jnp.int32, (_BR, 1), 0) + i * _BR
    Ym = jnp.where(rid < _N, Y, 0.0)

    @pl.when(i == 0)
    def _():
        s_ref[...] = jnp.zeros_like(s_ref)
        ss_ref[...] = jnp.zeros_like(ss_ref)

    s_ref[...] += jnp.sum(Ym, axis=0, keepdims=True)
    ss_ref[...] += jnp.sum(Ym * Ym, axis=0, keepdims=True)


def _k_out(yin, a, c, o):
    o[...] = jnp.maximum(yin[...] * a[...] + c[...], 0.0)


_ROWS = lambda: pl.BlockSpec((_BR, _D), lambda i: (i, 0))
_PAIR = lambda: pl.BlockSpec((2, _BR, _D), lambda i: (0, i, 0))
_WMAT = lambda: pl.BlockSpec((_D, _D), lambda i: (0, 0))
_VEC = lambda: pl.BlockSpec((1, _D), lambda i: (0, 0))

_MM_OUT = lambda: (
    [jax.ShapeDtypeStruct((_NP, _D), jnp.float32),
     jax.ShapeDtypeStruct((1, _D), jnp.float32),
     jax.ShapeDtypeStruct((1, _D), jnp.float32)],
    [_ROWS(), _VEC(), _VEC()],
)


def _call_agg_mm(P, x, w, b):
    out_shape, out_specs = _MM_OUT()
    return pl.pallas_call(
        _k_agg_mm, grid=(_GRID,),
        in_specs=[_PAIR(), _ROWS(), _WMAT(), _VEC()],
        out_specs=out_specs, out_shape=out_shape,
    )(P, x, w, b)


def _call_aff(yin, a, c, w, b):
    out_shape, out_specs = _MM_OUT()
    return pl.pallas_call(
        _k_aff, grid=(_GRID,),
        in_specs=[_ROWS(), _VEC(), _VEC(), _WMAT(), _VEC()],
        out_specs=out_specs, out_shape=out_shape,
    )(yin, a, c, w, b)


def _call_dual(P, x, Q, h, wa, wb, b):
    out_shape, out_specs = _MM_OUT()
    return pl.pallas_call(
        _k_dual, grid=(_GRID,),
        in_specs=[_PAIR(), _ROWS(), _PAIR(), _ROWS(), _WMAT(), _WMAT(),
                  _VEC()],
        out_specs=out_specs, out_shape=out_shape,
    )(P, x, Q, h, wa, wb, b)


def _call_out(yin, a, c):
    return pl.pallas_call(
        _k_out, grid=(_GRID,),
        in_specs=[_ROWS(), _VEC(), _VEC()],
        out_specs=_ROWS(),
        out_shape=jax.ShapeDtypeStruct((_NP, _D), jnp.float32),
    )(yin, a, c)


def _affine(s, ss, g, be):
    mean = s[0] / _N
    var = ss[0] / _N - mean * mean
    scale = g / jnp.sqrt(var + _EPS)
    shift = be - mean * scale
    return scale.reshape(1, _D), shift.reshape(1, _D)


def kernel(x, edge_index, params):
    p = params
    src = edge_index[0]
    dst = edge_index[1]

    xp = jnp.pad(x, ((0, _NP - _N), (0, 0)))
    epad = _NROWS * _EC - src.shape[0]
    src2 = jnp.concatenate([src, jnp.zeros((epad,), src.dtype)]
                           ).reshape(_NROWS, _EC)
    dst2 = jnp.concatenate([dst, jnp.full((epad,), _N, dst.dtype)]
                           ).reshape(_NROWS, _EC)

    b = lambda k: p[k].reshape(1, _D)

    P = _sc_agg(xp, src2, dst2)

    y1, s1, ss1 = _call_agg_mm(P, xp, p['W1'], b('b1'))
    sc1, sh1 = _affine(s1, ss1, p['g1'], p['be1'])
    y2, s2, ss2 = _call_aff(y1, sc1, sh1, p['W2'], b('b2'))
    sc2, sh2 = _affine(s2, ss2, p['g2'], p['be2'])
    h2 = _call_out(y2, sc2, sh2)

    Q = _sc_agg(h2, src2, dst2)

    y3, s3, ss3 = _call_dual(P, xp, Q, h2, p['W3'][:_D], p['W3'][_D:],
                             b('b3'))
    sc3, sh3 = _affine(s3, ss3, p['g3'], p['be3'])
    y4, s4, ss4 = _call_aff(y3, sc3, sh3, p['W4'], b('b4'))
    sc4, sh4 = _affine(s4, ss4, p['g4'], p['be4'])
    y5, s5, ss5 = _call_aff(y4, sc4, sh4, p['W5'], b('b5'))
    sc5, sh5 = _affine(s5, ss5, p['g5'], p['be5'])
    out = _call_out(y5, sc5, sh5)
    return out[:_N]


# speedup vs baseline: 1.0408x; 1.0408x over previous
import functools

import jax
import jax.numpy as jnp
from jax import lax
from jax.experimental import pallas as pl
from jax.experimental.pallas import tpu as pltpu
from jax.experimental.pallas import tpu_sc as plsc

_N = 10000
_D = 128
_NP = 10240
_BR = 1024
_GRID = _NP // _BR
_EPS = 1e-5

_EC = 128
_CH0 = 124
_CH1 = 36
_NB = 2
_NI = 2 * _NB
_EPAD = 16 * (_CH0 + _CH1) * _EC
_NROWS = _EPAD // _EC + _NI
_RPS = _NP // 16


def _sc_agg_body(h_hbm, src_hbm, dst_hbm, out_hbm, accum, sidx, didx, rows,
                 sem_g, sem_is, sem_id):
    c = lax.axis_index("c")
    s = lax.axis_index("s")

    pltpu.sync_copy(h_hbm.at[pl.ds(s * _RPS, _RPS)],
                    accum.at[pl.ds(s * _RPS, _RPS)])
    plsc.subcore_barrier()

    def _pipeline(ch, base):
        def _ld_idx(j, sl):
            return (pltpu.make_async_copy(src_hbm.at[base + j], sidx.at[sl],
                                          sem_is.at[sl]),
                    pltpu.make_async_copy(dst_hbm.at[base + j], didx.at[sl],
                                          sem_id.at[sl]))

        def _gather(sl, b):
            return pltpu.make_async_copy(h_hbm.at[sidx.at[sl]], rows.at[b],
                                         sem_g.at[b])

        for k in range(_NI):
            scp, dcp = _ld_idx(k, k)
            scp.start()
            dcp.start()

        for b in range(_NB):
            scp, _ = _ld_idx(b, b)
            scp.wait()
            _gather(b, b).start()

        def _visit(j, b, last):
            ib = j % _NI
            _gather(ib, b).wait()
            _, dcp = _ld_idx(j, ib)
            dcp.wait()
            pltpu.sync_copy(rows.at[b], accum.at[didx.at[ib]], add=True)
            scp, dcp = _ld_idx(j + _NI, ib)
            scp.start()
            dcp.start()
            if not last:
                jn = j + _NB
                scp2, _ = _ld_idx(jn, jn % _NI)
                scp2.wait()
                _gather(jn % _NI, b).start()

        def outer(grp, carry):
            for b in range(_NB):
                _visit(grp * _NB + b, b, False)
            return carry

        lax.fori_loop(0, ch // _NB - 1, outer, 0)
        for b in range(_NB):
            _visit((ch // _NB - 1) * _NB + b, b, True)

        for k in range(_NI):
            jo = ch + k
            scp, dcp = _ld_idx(jo, jo % _NI)
            scp.wait()
            dcp.wait()

    @pl.when(c == 0)
    def _():
        _pipeline(_CH0, s * _CH0)

    @pl.when(c != 0)
    def _():
        _pipeline(_CH1, 16 * _CH0 + s * _CH1)

    plsc.subcore_barrier()
    pltpu.sync_copy(accum.at[pl.ds(s * _RPS, _RPS)],
                    out_hbm.at[c, pl.ds(s * _RPS, _RPS)])


def _sc_agg(hp, src2, dst2):
    mesh = plsc.VectorSubcoreMesh(core_axis_name="c", subcore_axis_name="s")
    f = pl.kernel(
        _sc_agg_body, mesh=mesh,
        out_type=jax.ShapeDtypeStruct((2, _NP, _D), jnp.float32),
        scratch_types=[
            pltpu.VMEM_SHARED((_NP, _D), jnp.float32),
            pltpu.VMEM((_NI, _EC), jnp.int32),
            pltpu.VMEM((_NI, _EC), jnp.int32),
            pltpu.VMEM((_NB, _EC, _D), jnp.float32),
            pltpu.SemaphoreType.DMA((_NB,)),
            pltpu.SemaphoreType.DMA((_NI,)),
            pltpu.SemaphoreType.DMA((_NI,)),
        ],
    )
    return f(hp, src2, dst2)


def _mm_stats(X, w_ref, b_ref, y_ref, s_ref, ss_ref):
    Y = lax.dot_general(X, w_ref[...], (((1,), (0,)), ((), ())),
                        precision=lax.Precision.HIGHEST) + b_ref[...]
    y_ref[...] = Y
    i = pl.program_id(0)
    rid = lax.broadcasted_iota(jnp.int32, (_BR, 1), 0) + i * _BR
    Ym = jnp.where(rid < _N, Y, 0.0)

    @pl.when(i == 0)
    def _():
        s_ref[...] = jnp.zeros_like(s_ref)
        ss_ref[...] = jnp.zeros_like(ss_ref)

    s_ref[...] += jnp.sum(Ym, axis=0, keepdims=True)
    ss_ref[...] += jnp.sum(Ym * Ym, axis=0, keepdims=True)


def _k_agg_mm(p_ref, x_ref, w, b, y, s, ss):
    X = p_ref[0] + p_ref[1] - x_ref[...]
    _mm_stats(X, w, b, y, s, ss)


def _k_aff(yin, a, c, w, b, y, s, ss):
    X = jnp.maximum(yin[...] * a[...] + c[...], 0.0)
    _mm_stats(X, w, b, y, s, ss)


def _k_dual(p_ref, x_ref, q_ref, h_ref, wa, wb, b, y_ref, s_ref, ss_ref):
    X1 = p_ref[0] + p_ref[1] - x_ref[...]
    X2 = q_ref[0] + q_ref[1] - h_ref[...]
    Y = (lax.dot_general(X1, wa[...], (((1,), (0,)), ((), ())),
                         precision=lax.Precision.HIGHEST)
         + lax.dot_general(X2, wb[...], (((1,), (0,)), ((), ())),
                           precision=lax.Precision.HIGHEST) + b[...])
    y_ref[...] = Y
    i = pl.program_id(0)
    rid = lax.broadcasted_iota(jnp.int32, (_BR, 1), 0) + i * _BR
    Ym = jnp.where(rid < _N, Y, 0.0)

    @pl.when(i == 0)
    def _():
        s_ref[...] = jnp.zeros_like(s_ref)
        ss_ref[...] = jnp.zeros_like(ss_ref)

    s_ref[...] += jnp.sum(Ym, axis=0, keepdims=True)
    ss_ref[...] += jnp.sum(Ym * Ym, axis=0, keepdims=True)


def _k_out(yin, a, c, o):
    o[...] = jnp.maximum(yin[...] * a[...] + c[...], 0.0)


_ROWS = lambda: pl.BlockSpec((_BR, _D), lambda i: (i, 0))
_PAIR = lambda: pl.BlockSpec((2, _BR, _D), lambda i: (0, i, 0))
_WMAT = lambda: pl.BlockSpec((_D, _D), lambda i: (0, 0))
_VEC = lambda: pl.BlockSpec((1, _D), lambda i: (0, 0))

_MM_OUT = lambda: (
    [jax.ShapeDtypeStruct((_NP, _D), jnp.float32),
     jax.ShapeDtypeStruct((1, _D), jnp.float32),
     jax.ShapeDtypeStruct((1, _D), jnp.float32)],
    [_ROWS(), _VEC(), _VEC()],
)


def _call_agg_mm(P, x, w, b):
    out_shape, out_specs = _MM_OUT()
    return pl.pallas_call(
        _k_agg_mm, grid=(_GRID,),
        in_specs=[_PAIR(), _ROWS(), _WMAT(), _VEC()],
        out_specs=out_specs, out_shape=out_shape,
    )(P, x, w, b)


def _call_aff(yin, a, c, w, b):
    out_shape, out_specs = _MM_OUT()
    return pl.pallas_call(
        _k_aff, grid=(_GRID,),
        in_specs=[_ROWS(), _VEC(), _VEC(), _WMAT(), _VEC()],
        out_specs=out_specs, out_shape=out_shape,
    )(yin, a, c, w, b)


def _call_dual(P, x, Q, h, wa, wb, b):
    out_shape, out_specs = _MM_OUT()
    return pl.pallas_call(
        _k_dual, grid=(_GRID,),
        in_specs=[_PAIR(), _ROWS(), _PAIR(), _ROWS(), _WMAT(), _WMAT(),
                  _VEC()],
        out_specs=out_specs, out_shape=out_shape,
    )(P, x, Q, h, wa, wb, b)


def _call_out(yin, a, c):
    return pl.pallas_call(
        _k_out, grid=(_GRID,),
        in_specs=[_ROWS(), _VEC(), _VEC()],
        out_specs=_ROWS(),
        out_shape=jax.ShapeDtypeStruct((_NP, _D), jnp.float32),
    )(yin, a, c)


def _affine(s, ss, g, be):
    mean = s[0] / _N
    var = ss[0] / _N - mean * mean
    scale = g / jnp.sqrt(var + _EPS)
    shift = be - mean * scale
    return scale.reshape(1, _D), shift.reshape(1, _D)


def kernel(x, edge_index, params):
    p = params
    src = edge_index[0]
    dst = edge_index[1]

    xp = jnp.pad(x, ((0, _NP - _N), (0, 0)))
    epad = _NROWS * _EC - src.shape[0]
    src2 = jnp.concatenate([src, jnp.zeros((epad,), src.dtype)]
                           ).reshape(_NROWS, _EC)
    dst2 = jnp.concatenate([dst, jnp.full((epad,), _N, dst.dtype)]
                           ).reshape(_NROWS, _EC)

    b = lambda k: p[k].reshape(1, _D)

    P = _sc_agg(xp, src2, dst2)

    y1, s1, ss1 = _call_agg_mm(P, xp, p['W1'], b('b1'))
    sc1, sh1 = _affine(s1, ss1, p['g1'], p['be1'])
    y2, s2, ss2 = _call_aff(y1, sc1, sh1, p['W2'], b('b2'))
    sc2, sh2 = _affine(s2, ss2, p['g2'], p['be2'])
    h2 = _call_out(y2, sc2, sh2)

    Q = _sc_agg(h2, src2, dst2)

    y3, s3, ss3 = _call_dual(P, xp, Q, h2, p['W3'][:_D], p['W3'][_D:],
                             b('b3'))
    sc3, sh3 = _affine(s3, ss3, p['g3'], p['be3'])
    y4, s4, ss4 = _call_aff(y3, sc3, sh3, p['W4'], b('b4'))
    sc4, sh4 = _affine(s4, ss4, p['g4'], p['be4'])
    y5, s5, ss5 = _call_aff(y4, sc4, sh4, p['W5'], b('b5'))
    sc5, sh5 = _affine(s5, ss5, p['g5'], p['be5'])
    out = _call_out(y5, sc5, sh5)
    return out[:_N]
